# baseline (speedup 1.0000x reference)
import functools
import math

import jax
import jax.numpy as jnp
from jax import lax
from jax.experimental import pallas as pl
from jax.experimental.pallas import tpu as pltpu
from jax.sharding import Mesh, PartitionSpec as P

_VMEM_LIMIT_BYTES = 48 * 1024 * 1024
_SQRT_2_OVER_PI = math.sqrt(2.0 / math.pi)


def _gelu_tanh(x):
    return 0.5 * x * (1.0 + jnp.tanh(_SQRT_2_OVER_PI
                                     * (x + 0.044715 * x * x * x)))


def _fused_kernel(tok_row_ref, hid_ref, w_ref, b_ref, out_ref, *, n_bins):
    tok_row = tok_row_ref[0]
    hid = hid_ref[0].astype(jnp.bfloat16)
    s_len = hid.shape[0]

    iota_m = lax.broadcasted_iota(jnp.int32, (n_bins, s_len), 0)
    oh_mt = (iota_m == tok_row).astype(jnp.bfloat16)
    cell = jnp.dot(oh_mt, hid, preferred_element_type=jnp.float32)

    proj = lax.dot_general(cell.astype(jnp.bfloat16), w_ref[...],
                           (((1,), (1,)), ((), ())),
                           preferred_element_type=jnp.float32) + b_ref[...]
    cell2 = _gelu_tanh(proj).astype(jnp.bfloat16)

    out_ref[0] = lax.dot_general(oh_mt, cell2, (((0,), (0,)), ((), ())),
                                 preferred_element_type=jnp.float32)


def _projection_impl(hidden_states, token_type_ids, weight, bias):
    B, S, H = hidden_states.shape
    n_bins = 256

    wt = weight.astype(jnp.bfloat16)
    b2 = bias.reshape(1, H).astype(jnp.float32)
    tok_row = token_type_ids.astype(jnp.int32).reshape(B, 1, S)

    return pl.pallas_call(
        functools.partial(_fused_kernel, n_bins=n_bins),
        out_shape=jax.ShapeDtypeStruct((B, S, H), jnp.float32),
        grid=(B,),
        in_specs=[
            pl.BlockSpec((1, 1, S), lambda b: (b, 0, 0)),
            pl.BlockSpec((1, S, H), lambda b: (b, 0, 0)),
            pl.BlockSpec((H, H), lambda b: (0, 0)),
            pl.BlockSpec((1, H), lambda b: (0, 0)),
        ],
        out_specs=pl.BlockSpec((1, S, H), lambda b: (b, 0, 0)),
        compiler_params=pltpu.CompilerParams(
            dimension_semantics=("arbitrary",),
            vmem_limit_bytes=_VMEM_LIMIT_BYTES),
    )(tok_row, hidden_states, wt, b2)


def kernel(hidden_states, token_type_ids, weight, bias):
    B = hidden_states.shape[0]
    devs = [d for d in jax.devices() if d.platform == "tpu"]
    n_dev = len(devs)
    if n_dev >= 2 and B % 2 == 0:
        mesh = Mesh(devs[:2], ("d",))
        sharded = jax.shard_map(
            _projection_impl, mesh=mesh,
            in_specs=(P("d"), P("d"), P(), P()),
            out_specs=P("d"), check_vma=False)
        return sharded(hidden_states, token_type_ids, weight, bias)
    return _projection_impl(hidden_states, token_type_ids, weight, bias)


# speedup vs baseline: 6.9174x; 6.9174x over previous
import functools
import math

import jax
import jax.numpy as jnp
from jax import lax
from jax.experimental import pallas as pl
from jax.experimental.pallas import tpu as pltpu

_VMEM_LIMIT_BYTES = 48 * 1024 * 1024
_SQRT_2_OVER_PI = math.sqrt(2.0 / math.pi)


def _gelu_tanh(x):
    return 0.5 * x * (1.0 + jnp.tanh(_SQRT_2_OVER_PI
                                     * (x + 0.044715 * x * x * x)))


def _fused_kernel(tok_ref, hid_ref, w_ref, b_ref, out_ref, *, n_bins, n_batch):
    b_idx = pl.program_id(0)
    hid = hid_ref[0].astype(jnp.bfloat16)
    s_len = hid.shape[0]

    sub_iota = lax.broadcasted_iota(jnp.int32, (n_batch, 1), 0)
    tok_row = jnp.sum(jnp.where(sub_iota == b_idx, tok_ref[...], 0),
                      axis=0, keepdims=True)

    iota_m = lax.broadcasted_iota(jnp.int32, (n_bins, s_len), 0)
    oh_mt = (iota_m == tok_row).astype(jnp.bfloat16)
    cell = jnp.dot(oh_mt, hid, preferred_element_type=jnp.float32)

    proj = lax.dot_general(cell.astype(jnp.bfloat16),
                           w_ref[...].astype(jnp.bfloat16),
                           (((1,), (1,)), ((), ())),
                           preferred_element_type=jnp.float32) + b_ref[...][None, :]
    cell2 = _gelu_tanh(proj).astype(jnp.bfloat16)

    out_ref[0] = lax.dot_general(oh_mt, cell2, (((0,), (0,)), ((), ())),
                                 preferred_element_type=jnp.float32)


def kernel(hidden_states, token_type_ids, weight, bias):
    B, S, H = hidden_states.shape
    n_bins = 256

    return pl.pallas_call(
        functools.partial(_fused_kernel, n_bins=n_bins, n_batch=B),
        out_shape=jax.ShapeDtypeStruct((B, S, H), jnp.float32),
        grid=(B,),
        in_specs=[
            pl.BlockSpec((B, S), lambda b: (0, 0)),
            pl.BlockSpec((1, S, H), lambda b: (b, 0, 0)),
            pl.BlockSpec((H, H), lambda b: (0, 0)),
            pl.BlockSpec((H,), lambda b: (0,)),
        ],
        out_specs=pl.BlockSpec((1, S, H), lambda b: (b, 0, 0)),
        compiler_params=pltpu.CompilerParams(
            dimension_semantics=("arbitrary",),
            vmem_limit_bytes=_VMEM_LIMIT_BYTES),
    )(token_type_ids, hidden_states, weight, bias)


# speedup vs baseline: 7.0726x; 1.0224x over previous
import functools
import math

import jax
import jax.numpy as jnp
from jax import lax
from jax.experimental import pallas as pl
from jax.experimental.pallas import tpu as pltpu

_VMEM_LIMIT_BYTES = 64 * 1024 * 1024
_SQRT_2_OVER_PI = math.sqrt(2.0 / math.pi)


def _gelu_tanh(x):
    return 0.5 * x * (1.0 + jnp.tanh(_SQRT_2_OVER_PI
                                     * (x + 0.044715 * x * x * x)))


def _fused_kernel(tok_ref, hid_a_ref, hid_b_ref, w_ref, b_ref, out_ref,
                  *, n_bins, n_batch):
    b_idx = pl.program_id(0)
    s_half = hid_a_ref.shape[1]
    s_len = 2 * s_half

    sub_iota = lax.broadcasted_iota(jnp.int32, (n_batch, 1), 0)
    tok_row = jnp.sum(jnp.where(sub_iota == b_idx, tok_ref[...], 0),
                      axis=0, keepdims=True)

    iota_m = lax.broadcasted_iota(jnp.int32, (n_bins, s_len), 0)
    oh_mt = (iota_m == tok_row).astype(jnp.bfloat16)

    cell = (jnp.dot(oh_mt[:, :s_half], hid_a_ref[0].astype(jnp.bfloat16),
                    preferred_element_type=jnp.float32)
            + jnp.dot(oh_mt[:, s_half:], hid_b_ref[0].astype(jnp.bfloat16),
                      preferred_element_type=jnp.float32))

    proj = lax.dot_general(cell.astype(jnp.bfloat16),
                           w_ref[...].astype(jnp.bfloat16),
                           (((1,), (1,)), ((), ())),
                           preferred_element_type=jnp.float32) + b_ref[...][None, :]
    cell2 = _gelu_tanh(proj).astype(jnp.bfloat16)

    out_ref[0] = lax.dot_general(oh_mt, cell2, (((0,), (0,)), ((), ())),
                                 preferred_element_type=jnp.float32)


def kernel(hidden_states, token_type_ids, weight, bias):
    B, S, H = hidden_states.shape
    n_bins = 256
    S2 = S // 2

    return pl.pallas_call(
        functools.partial(_fused_kernel, n_bins=n_bins, n_batch=B),
        out_shape=jax.ShapeDtypeStruct((B, S, H), jnp.float32),
        grid=(B,),
        in_specs=[
            pl.BlockSpec((B, S), lambda b: (0, 0)),
            pl.BlockSpec((1, S2, H), lambda b: (b, 0, 0)),
            pl.BlockSpec((1, S2, H), lambda b: (b, 1, 0)),
            pl.BlockSpec((H, H), lambda b: (0, 0)),
            pl.BlockSpec((H,), lambda b: (0,)),
        ],
        out_specs=pl.BlockSpec((1, S, H), lambda b: (b, 0, 0)),
        compiler_params=pltpu.CompilerParams(
            dimension_semantics=("arbitrary",),
            vmem_limit_bytes=_VMEM_LIMIT_BYTES),
    )(token_type_ids, hidden_states, hidden_states, weight, bias)
